# baseline (speedup 1.0000x reference)
import jax
import jax.numpy as jnp
from jax.experimental import pallas as pl
import jax.experimental.pallas.tpu as pltpu

B, S, E, KD, VD, H, O = 64, 2048, 1024, 512, 512, 16, 20
DH = E // H
F32 = jnp.float32


NSTR = 8
SC_ = S // NSTR
BN = 4


def _fused_body(x_ref, Wlin_ref, blin_ref, Wq_ref, bq_ref, Wk_ref,
                amask_ref, kpm_ref, *rest):
    (seq_refs, (Wv_ref, bv_ref, Wo_ref, bo_ref, Wla_ref, bla_ref,
                g_ref, mask_ref, onehot_ref, pred_ref,
                p_scr, u_scr, pred0_scr)) = rest[:NSTR], rest[NSTR:]
    b = pl.program_id(0)

    @pl.when(b == 0)
    def _setup():
        x = x_ref[...]
        pred0_scr[...] = (
            jnp.dot(x, Wlin_ref[...].T, preferred_element_type=F32)
            + blin_ref[...])
        q = (jnp.dot(x, Wq_ref[...].T, preferred_element_type=F32)
             + bq_ref[...])
        q = q * jnp.float32(1.0 / 8.0)
        for h in range(H):
            qh = q[:, h * DH:(h + 1) * DH]
            wk_h = Wk_ref[h * DH:(h + 1) * DH, :]
            p_scr[:, h, :] = jnp.dot(qh, wk_h, preferred_element_type=F32)

    for j in range(BN):
        row = b * BN + j
        seq_bfs = [r[j, 0].astype(jnp.bfloat16) for r in seq_refs]
        pb = p_scr[pl.ds(row, 1), :, :][0].astype(jnp.bfloat16)
        dn = (((1,), (1,)), ((), ()))
        scores = jnp.concatenate(
            [jax.lax.dot_general(sb, pb, dn, preferred_element_type=F32)
             for sb in seq_bfs], axis=0)
        am = amask_ref[j, 0, :]
        kpm = kpm_ref[j, 0, :]
        scores = scores + am[:, None]
        scores = jnp.where(kpm[:, None] > 0, jnp.float32(-1e9), scores)
        m = jnp.max(scores, axis=0, keepdims=True)
        e = jnp.exp(scores - m)
        l = jnp.sum(e, axis=0, keepdims=True)
        a = (e / l).astype(jnp.bfloat16)
        dn0 = (((0,), (0,)), ((), ()))
        u = sum(jax.lax.dot_general(a[i * SC_:(i + 1) * SC_], seq_bfs[i],
                                    dn0, preferred_element_type=F32)
                for i in range(NSTR))
        u_scr[pl.ds(row, 1), :, :] = u[None]

    @pl.when(b == B // BN - 1)
    def _final():
        ctx_parts = []
        for h in range(H):
            uh = u_scr[:, h, :]
            wv_h = Wv_ref[h * DH:(h + 1) * DH, :]
            ctx_parts.append(jax.lax.dot_general(
                uh, wv_h, (((1,), (1,)), ((), ())),
                preferred_element_type=F32))
        ctx = jnp.concatenate(ctx_parts, axis=1) + bv_ref[...]
        hout = (jnp.dot(ctx, Wo_ref[...].T, preferred_element_type=F32)
                + bo_ref[...])
        pred = (pred0_scr[...]
                + jnp.dot(hout, Wla_ref[...].T, preferred_element_type=F32)
                + bla_ref[...])
        y = pred + g_ref[...]
        top = jnp.argmax(y, axis=-1)
        onehot = (jax.lax.broadcasted_iota(jnp.int32, (B, O), 1)
                  == top[:, None]).astype(F32)
        msk = mask_ref[...]
        pred_ref[...] = pred * msk
        onehot_ref[...] = onehot * msk


def kernel(node_emb, seq_emb, mask, key_padding_mask, attn_mask,
           W_lin, b_lin, Wq, Wk, Wv, bq, bk, bv, Wo, bo, W_la, b_la):
    del bk
    x = node_emb.reshape(B, E)
    kpm_f = key_padding_mask.astype(F32).reshape(B, 1, S)
    amask = attn_mask.reshape(B, 1, S)
    mask2d = mask.reshape(B, 1)
    skey = jax.random.fold_in(jax.random.key(0), 12345)
    g = jax.random.gumbel(skey, (B, O), F32)

    const = lambda b: (0, 0)
    onehot, pred = pl.pallas_call(
        _fused_body,
        grid=(B // BN,),
        in_specs=[
            pl.BlockSpec((B, E), const),
            pl.BlockSpec((O, E), const),
            pl.BlockSpec((1, O), const),
            pl.BlockSpec((E, E), const),
            pl.BlockSpec((1, E), const),
            pl.BlockSpec((E, KD), const),
            pl.BlockSpec((BN, 1, S), lambda b: (b, 0, 0)),
            pl.BlockSpec((BN, 1, S), lambda b: (b, 0, 0)),
        ] + [
            pl.BlockSpec((BN, 1, SC_, KD),
                         (lambda i: (lambda b: (b, i, 0, 0)))(i))
            for i in range(NSTR)
        ] + [
            pl.BlockSpec((E, KD), const),
            pl.BlockSpec((1, E), const),
            pl.BlockSpec((E, E), const),
            pl.BlockSpec((1, E), const),
            pl.BlockSpec((O, E), const),
            pl.BlockSpec((1, O), const),
            pl.BlockSpec((B, O), const),
            pl.BlockSpec((B, 1), const),
        ],
        out_specs=(
            pl.BlockSpec((B, O), const),
            pl.BlockSpec((B, O), const),
        ),
        out_shape=(
            jax.ShapeDtypeStruct((B, O), F32),
            jax.ShapeDtypeStruct((B, O), F32),
        ),
        compiler_params=pltpu.CompilerParams(
            vmem_limit_bytes=110 * 1024 * 1024),
        scratch_shapes=[
            pltpu.VMEM((B, H, KD), F32),
            pltpu.VMEM((B, H, KD), F32),
            pltpu.VMEM((B, O), F32),
        ],
    )(x, W_lin, b_lin.reshape(1, O), Wq, bq.reshape(1, E), Wk,
      amask, kpm_f,
      *([seq_emb.reshape(B, NSTR, SC_, KD)] * NSTR),
      Wv, bv.reshape(1, E), Wo, bo.reshape(1, E), W_la, b_la.reshape(1, O),
      g, mask2d)

    return (onehot, pred)


# speedup vs baseline: 1.1047x; 1.1047x over previous
import jax
import jax.numpy as jnp
from jax.experimental import pallas as pl
import jax.experimental.pallas.tpu as pltpu

B, S, E, KD, VD, H, O = 64, 2048, 1024, 512, 512, 16, 20
DH = E // H
F32 = jnp.float32


NSTR = 4
SC_ = S // NSTR
BN = 4


def _fused_body(x_ref, Wlin_ref, blin_ref, Wq_ref, bq_ref, Wk_ref,
                amask_ref, kpm_ref, *rest):
    (seq_refs, (Wv_ref, bv_ref, Wo_ref, bo_ref, Wla_ref, bla_ref,
                g_ref, mask_ref, onehot_ref, pred_ref,
                p_scr, u_scr, pred0_scr)) = rest[:NSTR], rest[NSTR:]
    b = pl.program_id(0)

    @pl.when(b == 0)
    def _setup():
        x = x_ref[...]
        pred0_scr[...] = (
            jnp.dot(x, Wlin_ref[...].T, preferred_element_type=F32)
            + blin_ref[...])
        q = (jnp.dot(x, Wq_ref[...].T, preferred_element_type=F32)
             + bq_ref[...])
        q = q * jnp.float32(1.0 / 8.0)
        for h in range(H):
            qh = q[:, h * DH:(h + 1) * DH]
            wk_h = Wk_ref[h * DH:(h + 1) * DH, :]
            p_scr[:, h, :] = jnp.dot(qh, wk_h, preferred_element_type=F32)

    for j in range(BN):
        row = b * BN + j
        seq_bfs = [r[j, 0].astype(jnp.bfloat16) for r in seq_refs]
        pb = p_scr[pl.ds(row, 1), :, :][0].astype(jnp.bfloat16)
        dn = (((1,), (1,)), ((), ()))
        scores = jnp.concatenate(
            [jax.lax.dot_general(pb, sb, dn, preferred_element_type=F32)
             for sb in seq_bfs], axis=1)
        am = amask_ref[j, :, :]
        kpm = kpm_ref[j, :, :]
        scores = scores + am
        scores = jnp.where(kpm > 0, jnp.float32(-1e9), scores)
        m = jnp.max(scores, axis=1, keepdims=True)
        e = (jnp.exp(scores - m)).astype(jnp.bfloat16)
        l = jnp.sum(e.astype(F32), axis=1, keepdims=True)
        dn1 = (((1,), (0,)), ((), ()))
        u = sum(jax.lax.dot_general(e[:, i * SC_:(i + 1) * SC_], seq_bfs[i],
                                    dn1, preferred_element_type=F32)
                for i in range(NSTR))
        u_scr[pl.ds(row, 1), :, :] = (u / l)[None]

    @pl.when(b == B // BN - 1)
    def _final():
        ctx_parts = []
        for h in range(H):
            uh = u_scr[:, h, :]
            wv_h = Wv_ref[h * DH:(h + 1) * DH, :]
            ctx_parts.append(jax.lax.dot_general(
                uh, wv_h, (((1,), (1,)), ((), ())),
                preferred_element_type=F32))
        ctx = jnp.concatenate(ctx_parts, axis=1) + bv_ref[...]
        hout = (jnp.dot(ctx, Wo_ref[...].T, preferred_element_type=F32)
                + bo_ref[...])
        pred = (pred0_scr[...]
                + jnp.dot(hout, Wla_ref[...].T, preferred_element_type=F32)
                + bla_ref[...])
        y = pred + g_ref[...]
        top = jnp.argmax(y, axis=-1)
        onehot = (jax.lax.broadcasted_iota(jnp.int32, (B, O), 1)
                  == top[:, None]).astype(F32)
        msk = mask_ref[...]
        pred_ref[...] = pred * msk
        onehot_ref[...] = onehot * msk


def kernel(node_emb, seq_emb, mask, key_padding_mask, attn_mask,
           W_lin, b_lin, Wq, Wk, Wv, bq, bk, bv, Wo, bo, W_la, b_la):
    del bk
    x = node_emb.reshape(B, E)
    kpm_f = key_padding_mask.astype(F32).reshape(B, 1, S)
    amask = attn_mask.reshape(B, 1, S)
    mask2d = mask.reshape(B, 1)
    skey = jax.random.fold_in(jax.random.key(0), 12345)
    g = jax.random.gumbel(skey, (B, O), F32)

    const = lambda b: (0, 0)
    onehot, pred = pl.pallas_call(
        _fused_body,
        grid=(B // BN,),
        in_specs=[
            pl.BlockSpec((B, E), const),
            pl.BlockSpec((O, E), const),
            pl.BlockSpec((1, O), const),
            pl.BlockSpec((E, E), const),
            pl.BlockSpec((1, E), const),
            pl.BlockSpec((E, KD), const),
            pl.BlockSpec((BN, 1, S), lambda b: (b, 0, 0)),
            pl.BlockSpec((BN, 1, S), lambda b: (b, 0, 0)),
        ] + [
            pl.BlockSpec((BN, 1, SC_, KD),
                         (lambda i: (lambda b: (b, i, 0, 0)))(i))
            for i in range(NSTR)
        ] + [
            pl.BlockSpec((E, KD), const),
            pl.BlockSpec((1, E), const),
            pl.BlockSpec((E, E), const),
            pl.BlockSpec((1, E), const),
            pl.BlockSpec((O, E), const),
            pl.BlockSpec((1, O), const),
            pl.BlockSpec((B, O), const),
            pl.BlockSpec((B, 1), const),
        ],
        out_specs=(
            pl.BlockSpec((B, O), const),
            pl.BlockSpec((B, O), const),
        ),
        out_shape=(
            jax.ShapeDtypeStruct((B, O), F32),
            jax.ShapeDtypeStruct((B, O), F32),
        ),
        compiler_params=pltpu.CompilerParams(
            vmem_limit_bytes=110 * 1024 * 1024),
        scratch_shapes=[
            pltpu.VMEM((B, H, KD), F32),
            pltpu.VMEM((B, H, KD), F32),
            pltpu.VMEM((B, O), F32),
        ],
    )(x, W_lin, b_lin.reshape(1, O), Wq, bq.reshape(1, E), Wk,
      amask, kpm_f,
      *([seq_emb.reshape(B, NSTR, SC_, KD)] * NSTR),
      Wv, bv.reshape(1, E), Wo, bo.reshape(1, E), W_la, b_la.reshape(1, O),
      g, mask2d)

    return (onehot, pred)


# speedup vs baseline: 1.1195x; 1.0134x over previous
import jax
import jax.numpy as jnp
from jax.experimental import pallas as pl
import jax.experimental.pallas.tpu as pltpu

B, S, E, KD, VD, H, O = 64, 2048, 1024, 512, 512, 16, 20
DH = E // H
F32 = jnp.float32


NSTR = 4
SC_ = S // NSTR
BN = 4


def _fused_body(x_ref, Wlin_ref, blin_ref, Wq_ref, bq_ref, Wk_ref,
                amask_ref, kpm_ref, *rest):
    (seq_refs, (Wv_ref, bv_ref, Wo_ref, bo_ref, Wla_ref, bla_ref,
                g_ref, mask_ref, onehot_ref, pred_ref,
                p_scr, u_scr, pred0_scr)) = rest[:NSTR], rest[NSTR:]
    b = pl.program_id(0)

    @pl.when(b == 0)
    def _setup():
        x = x_ref[...]
        pred0_scr[...] = (
            jnp.dot(x, Wlin_ref[...].T, preferred_element_type=F32)
            + blin_ref[...])
        q = (jnp.dot(x, Wq_ref[...].T, preferred_element_type=F32)
             + bq_ref[...])
        q = q * jnp.float32(1.0 / 8.0)
        for h in range(H):
            qh = q[:, h * DH:(h + 1) * DH]
            wk_h = Wk_ref[h * DH:(h + 1) * DH, :]
            p_scr[:, h, :] = jnp.dot(qh, wk_h, preferred_element_type=F32)

    for j in range(BN):
        row = b * BN + j
        seqs = [r[j, 0] for r in seq_refs]
        pb = p_scr[pl.ds(row, 1), :, :][0]
        dn = (((1,), (1,)), ((), ()))
        scores = jnp.concatenate(
            [jax.lax.dot_general(pb, s, dn, preferred_element_type=F32)
             for s in seqs], axis=1)
        am = amask_ref[j, :, :]
        kpm = kpm_ref[j, :, :]
        scores = scores + am
        scores = jnp.where(kpm > 0, jnp.float32(-1e9), scores)
        m = jnp.max(scores, axis=1, keepdims=True)
        e = (jnp.exp(scores - m)).astype(jnp.bfloat16)
        l = jnp.sum(e.astype(F32), axis=1, keepdims=True)
        dn1 = (((1,), (0,)), ((), ()))
        u = sum(jax.lax.dot_general(e[:, i * SC_:(i + 1) * SC_],
                                    seqs[i].astype(jnp.bfloat16),
                                    dn1, preferred_element_type=F32)
                for i in range(NSTR))
        u_scr[pl.ds(row, 1), :, :] = (u / l)[None]

    @pl.when(b == B // BN - 1)
    def _final():
        ctx_parts = []
        for h in range(H):
            uh = u_scr[:, h, :]
            wv_h = Wv_ref[h * DH:(h + 1) * DH, :]
            ctx_parts.append(jax.lax.dot_general(
                uh, wv_h, (((1,), (1,)), ((), ())),
                preferred_element_type=F32))
        ctx = jnp.concatenate(ctx_parts, axis=1) + bv_ref[...]
        hout = (jnp.dot(ctx, Wo_ref[...].T, preferred_element_type=F32)
                + bo_ref[...])
        pred = (pred0_scr[...]
                + jnp.dot(hout, Wla_ref[...].T, preferred_element_type=F32)
                + bla_ref[...])
        y = pred + g_ref[...]
        top = jnp.argmax(y, axis=-1)
        onehot = (jax.lax.broadcasted_iota(jnp.int32, (B, O), 1)
                  == top[:, None]).astype(F32)
        msk = mask_ref[...]
        pred_ref[...] = pred * msk
        onehot_ref[...] = onehot * msk


def kernel(node_emb, seq_emb, mask, key_padding_mask, attn_mask,
           W_lin, b_lin, Wq, Wk, Wv, bq, bk, bv, Wo, bo, W_la, b_la):
    del bk
    x = node_emb.reshape(B, E)
    kpm_f = key_padding_mask.astype(F32).reshape(B, 1, S)
    amask = attn_mask.reshape(B, 1, S)
    mask2d = mask.reshape(B, 1)
    skey = jax.random.fold_in(jax.random.key(0), 12345)
    g = jax.random.gumbel(skey, (B, O), F32)

    const = lambda b: (0, 0)
    onehot, pred = pl.pallas_call(
        _fused_body,
        grid=(B // BN,),
        in_specs=[
            pl.BlockSpec((B, E), const),
            pl.BlockSpec((O, E), const),
            pl.BlockSpec((1, O), const),
            pl.BlockSpec((E, E), const),
            pl.BlockSpec((1, E), const),
            pl.BlockSpec((E, KD), const),
            pl.BlockSpec((BN, 1, S), lambda b: (b, 0, 0)),
            pl.BlockSpec((BN, 1, S), lambda b: (b, 0, 0)),
        ] + [
            pl.BlockSpec((BN, 1, SC_, KD),
                         (lambda i: (lambda b: (b, i, 0, 0)))(i))
            for i in range(NSTR)
        ] + [
            pl.BlockSpec((E, KD), const),
            pl.BlockSpec((1, E), const),
            pl.BlockSpec((E, E), const),
            pl.BlockSpec((1, E), const),
            pl.BlockSpec((O, E), const),
            pl.BlockSpec((1, O), const),
            pl.BlockSpec((B, O), const),
            pl.BlockSpec((B, 1), const),
        ],
        out_specs=(
            pl.BlockSpec((B, O), const),
            pl.BlockSpec((B, O), const),
        ),
        out_shape=(
            jax.ShapeDtypeStruct((B, O), F32),
            jax.ShapeDtypeStruct((B, O), F32),
        ),
        compiler_params=pltpu.CompilerParams(
            vmem_limit_bytes=110 * 1024 * 1024),
        scratch_shapes=[
            pltpu.VMEM((B, H, KD), F32),
            pltpu.VMEM((B, H, KD), F32),
            pltpu.VMEM((B, O), F32),
        ],
    )(x, W_lin, b_lin.reshape(1, O), Wq, bq.reshape(1, E), Wk,
      amask, kpm_f,
      *([seq_emb.reshape(B, NSTR, SC_, KD)] * NSTR),
      Wv, bv.reshape(1, E), Wo, bo.reshape(1, E), W_la, b_la.reshape(1, O),
      g, mask2d)

    return (onehot, pred)
